# baseline (speedup 1.0000x reference)
import functools

import jax
import jax.numpy as jnp
from jax import lax
from jax.experimental import pallas as pl
from jax.experimental.pallas import tpu as pltpu
from jax.experimental.pallas import tpu_sc as plsc

B = 16384
DIM = 32
NC = 2
NS = 16
L = 16
NW = NC * NS
BPW = B // NW
GCHUNK = 128
NG = BPW // GCHUNK
NCH = BPW // L

NBT = (1_000_000 + GCHUNK - 1) // GCHUNK
RPW = (NBT + NW - 1) // NW
WBLK = 10
WW = WBLK * GCHUNK
FCH = 2048
NFCH = B // FCH
NDUMP = L
UROWS = B + NDUMP

_mesh = plsc.VectorSubcoreMesh(
    core_axis_name="c", subcore_axis_name="s", num_cores=NC, num_subcores=NS
)

_sc_params = pltpu.CompilerParams(
    needs_layout_passes=False, use_tc_tiling_on_sc=True
)


@functools.partial(
    pl.kernel,
    out_type=(
        jax.ShapeDtypeStruct((UROWS, GCHUNK), jnp.float32),
        jax.ShapeDtypeStruct((UROWS, GCHUNK), jnp.float32),
    ),
    mesh=_mesh,
    scratch_types=[
        pltpu.VMEM((FCH,), jnp.int32),
        pltpu.VMEM((B + 4 * L,), jnp.int32),
        pltpu.VMEM((B + L,), jnp.int32),
        pltpu.VMEM((2, DIM, WW), jnp.float32),
        pltpu.VMEM((2, L, GCHUNK), jnp.float32),
        pltpu.SemaphoreType.DMA,
        pltpu.SemaphoreType.DMA,
        pltpu.SemaphoreType.DMA,
    ],
    compiler_params=_sc_params,
)
def _extract_kernel(users_hbm, items_hbm, ult_hbm, ilt_hbm,
                    u_out, i_out, stage_v, pv_v, wl_v,
                    win_v, rows_v, gsem, ssem0, ssem1):
    wid = lax.axis_index("s") * NC + lax.axis_index("c")
    lo_c = wid * RPW
    hi_c = jnp.minimum(lo_c + RPW, NBT)
    nwin = (hi_c - lo_c + WBLK - 1) // WBLK
    lo_u = lo_c * GCHUNK

    iota = lax.iota(jnp.int32, L)
    dump = jnp.int32(B) + iota

    for idx_hbm, tbl, dst in ((users_hbm, ult_hbm, u_out),
                              (items_hbm, ilt_hbm, i_out)):
        def wstart(w, buf, tbl=tbl):
            c0 = jnp.minimum(lo_c + w * WBLK, NBT - WBLK)
            off = pl.multiple_of(c0 * GCHUNK, GCHUNK)
            pltpu.async_copy(tbl.at[:, pl.ds(off, WW)], win_v.at[buf], gsem)

        wstart(0, 0)

        def f1outer(ch, n, idx_hbm=idx_hbm, iota=iota):
            src = lax.rem(wid + ch, NFCH) * FCH
            pltpu.sync_copy(idx_hbm.at[pl.ds(src, FCH)], stage_v)

            def f1(k, n, src=src):
                u = stage_v[pl.ds(k * L, L)]
                c = jax.lax.shift_right_logical(u, 7)
                m = (c >= lo_c) & (c < hi_c)
                b = src + k * L + iota
                packed = jax.lax.shift_left(u - lo_u, 14) | b
                plsc.store_compressed(pv_v.at[pl.ds(n, L)], packed, mask=m)
                return n + plsc.all_reduce_population_count(m)[0]

            return lax.fori_loop(0, FCH // L, f1, n, unroll=4)

        n = lax.fori_loop(0, NFCH, f1outer, jnp.int32(0))
        for q in range(4):
            pv_v[pl.ds(n + q * L, L)] = jnp.full((L,), -1, jnp.int32)
        nv4 = (n + 4 * L - 1) // (4 * L)

        def win_body(w, carry, tbl=tbl, dst=dst, nv4=nv4, iota=iota, dump=dump):
            pend0, pend1 = carry
            buf = lax.rem(w, 2)

            @pl.when(w + 1 < nwin)
            def _():
                wstart(w + 1, 1 - buf)

            wr0 = w * WBLK * GCHUNK

            def f2(k, wn):
                for q in range(4):
                    pv = pv_v[pl.ds(k * 4 * L + q * L, L)]
                    ur = jax.lax.shift_right_logical(pv, 14)
                    m = (pv >= 0) & (ur >= wr0) & (ur < wr0 + WW)
                    plsc.store_compressed(wl_v.at[pl.ds(wn, L)], pv, mask=m)
                    wn = wn + plsc.all_reduce_population_count(m)[0]
                return wn

            wn = lax.fori_loop(0, nv4, f2, jnp.int32(0))
            wl_v[pl.ds(wn, L)] = jnp.full((L,), -1, jnp.int32)
            ngrp = (wn + L - 1) // L

            pltpu.make_async_copy(
                tbl.at[:, pl.ds(0, WW)], win_v.at[0], gsem).wait()

            c0 = jnp.minimum(lo_c + w * WBLK, NBT - WBLK)
            wloc0 = (c0 - lo_c) * GCHUNK

            def g_body(g, pends, buf=buf, wloc0=wloc0, dst=dst):
                p0, p1 = pends
                rbuf = lax.rem(g, 2)
                pv = wl_v[pl.ds(g * L, L)]
                bvec = pv & 0x3FFF
                ur = jax.lax.shift_right_logical(pv, 14)
                bsafe = jnp.where(pv < 0, dump, bvec)
                ulocal = jnp.where(pv < 0, 0, ur - wloc0)

                @pl.when((rbuf == 0) & (p0 > 0))
                def _():
                    pltpu.make_async_copy(
                        dst.at[pl.ds(0, L)], rows_v.at[0], ssem0).wait()

                @pl.when((rbuf == 1) & (p1 > 0))
                def _():
                    pltpu.make_async_copy(
                        dst.at[pl.ds(0, L)], rows_v.at[1], ssem1).wait()

                for d in range(DIM):
                    dvec = jnp.full((L,), d, jnp.int32)
                    vals = plsc.load_gather(win_v.at[buf], [dvec, ulocal])
                    plsc.store_scatter(rows_v.at[rbuf], [iota, dvec], vals)

                @pl.when(rbuf == 0)
                def _():
                    pltpu.async_copy(rows_v.at[0], dst.at[bsafe], ssem0)

                @pl.when(rbuf == 1)
                def _():
                    pltpu.async_copy(rows_v.at[1], dst.at[bsafe], ssem1)

                new_p0 = jnp.where(rbuf == 0, jnp.int32(1), p0)
                new_p1 = jnp.where(rbuf == 1, jnp.int32(1), p1)
                return (new_p0, new_p1)

            return lax.fori_loop(0, ngrp, g_body, (pend0, pend1))

        pend0, pend1 = lax.fori_loop(
            0, nwin, win_body, (jnp.int32(0), jnp.int32(0)))

        @pl.when(pend0 > 0)
        def _():
            pltpu.make_async_copy(
                dst.at[pl.ds(0, L)], rows_v.at[0], ssem0).wait()

        @pl.when(pend1 > 0)
        def _():
            pltpu.make_async_copy(
                dst.at[pl.ds(0, L)], rows_v.at[1], ssem1).wait()


@functools.partial(
    pl.kernel,
    out_type=jax.ShapeDtypeStruct((B,), jnp.float32),
    mesh=_mesh,
    scratch_types=[
        pltpu.VMEM((BPW,), jnp.int32),
        pltpu.VMEM((BPW,), jnp.int32),
        pltpu.VMEM((2, GCHUNK, GCHUNK), jnp.float32),
        pltpu.VMEM((2, GCHUNK, GCHUNK), jnp.float32),
        pltpu.VMEM((BPW,), jnp.float32),
        pltpu.VMEM((BPW,), jnp.float32),
        pltpu.VMEM((BPW,), jnp.float32),
        pltpu.SemaphoreType.DMA,
        pltpu.SemaphoreType.DMA,
    ],
    compiler_params=_sc_params,
)
def _dot_kernel(users_hbm, items_hbm, u_hbm, i_hbm, ub_hbm, ib_hbm,
                out_hbm, uidx_v, iidx_v, usub_v, isub_v,
                ub_v, ib_v, out_v, sem, bsem):
    wid = lax.axis_index("s") * NC + lax.axis_index("c")
    base = wid * BPW
    iota = lax.iota(jnp.int32, L)

    pltpu.sync_copy(users_hbm.at[pl.ds(base, BPW)], uidx_v)
    pltpu.sync_copy(items_hbm.at[pl.ds(base, BPW)], iidx_v)

    bias_copies = []
    for g in range(NG):
        sl = pl.ds(g * GCHUNK, GCHUNK)
        bias_copies.append(
            pltpu.async_copy(ub_hbm.at[uidx_v.at[sl]], ub_v.at[sl], bsem))
        bias_copies.append(
            pltpu.async_copy(ib_hbm.at[iidx_v.at[sl]], ib_v.at[sl], bsem))

    def sstart(s, buf):
        sl = pl.ds(base + s * GCHUNK, GCHUNK)
        pltpu.async_copy(u_hbm.at[sl], usub_v.at[buf], sem)
        pltpu.async_copy(i_hbm.at[sl], isub_v.at[buf], sem)

    sstart(0, 0)
    for c in bias_copies:
        c.wait()

    def sub_body(s, _):
        buf = lax.rem(s, 2)
        pltpu.make_async_copy(
            u_hbm.at[pl.ds(0, GCHUNK)], usub_v.at[0], sem).wait()
        pltpu.make_async_copy(
            i_hbm.at[pl.ds(0, GCHUNK)], isub_v.at[0], sem).wait()

        @pl.when(s + 1 < NG)
        def _():
            sstart(s + 1, 1 - buf)

        def chunk(cc, _, buf=buf, s=s):
            j0 = s * GCHUNK + cc * L
            rows = cc * L + iota
            acc = ub_v[pl.ds(j0, L)] + ib_v[pl.ds(j0, L)]
            for d in range(DIM):
                dvec = jnp.full((L,), d, jnp.int32)
                uv = plsc.load_gather(usub_v.at[buf], [rows, dvec])
                iv = plsc.load_gather(isub_v.at[buf], [rows, dvec])
                acc = acc + uv * iv
            out_v[pl.ds(j0, L)] = acc
            return ()

        lax.fori_loop(0, GCHUNK // L, chunk, ())
        return ()

    lax.fori_loop(0, NG, sub_body, ())

    pltpu.sync_copy(out_v, out_hbm.at[pl.ds(base, BPW)])


def kernel(users, items, user_latent, item_latent, user_bias, item_bias):
    users = users.astype(jnp.int32)
    items = items.astype(jnp.int32)
    ub = user_bias.reshape(-1)
    ib = item_bias.reshape(-1)
    u_rows, i_rows = _extract_kernel(users, items, user_latent.T, item_latent.T)
    return _dot_kernel(users, items, u_rows, i_rows, ub, ib)


# speedup vs baseline: 1.0188x; 1.0188x over previous
import functools

import jax
import jax.numpy as jnp
from jax import lax
from jax.experimental import pallas as pl
from jax.experimental.pallas import tpu as pltpu
from jax.experimental.pallas import tpu_sc as plsc

B = 16384
DIM = 32
NC = 2
NS = 16
L = 16
NW = NC * NS
BPW = B // NW
GCHUNK = 128
NG = BPW // GCHUNK
NCH = BPW // L

NBT = (1_000_000 + GCHUNK - 1) // GCHUNK
RPW = (NBT + NW - 1) // NW
WBLK = 11
WW = WBLK * GCHUNK
FCH = 2048
NFCH = B // FCH
NDUMP = L
UROWS = B + NDUMP

_mesh = plsc.VectorSubcoreMesh(
    core_axis_name="c", subcore_axis_name="s", num_cores=NC, num_subcores=NS
)

_sc_params = pltpu.CompilerParams(
    needs_layout_passes=False, use_tc_tiling_on_sc=True
)


@functools.partial(
    pl.kernel,
    out_type=(
        jax.ShapeDtypeStruct((UROWS, GCHUNK), jnp.float32),
        jax.ShapeDtypeStruct((UROWS, GCHUNK), jnp.float32),
    ),
    mesh=_mesh,
    scratch_types=[
        pltpu.VMEM((FCH,), jnp.int32),
        pltpu.VMEM((B + 4 * L,), jnp.int32),
        pltpu.VMEM((B + L,), jnp.int32),
        pltpu.VMEM((2, DIM, WW), jnp.float32),
        pltpu.VMEM((2, L, GCHUNK), jnp.float32),
        pltpu.SemaphoreType.DMA,
        pltpu.SemaphoreType.DMA,
        pltpu.SemaphoreType.DMA,
    ],
    compiler_params=_sc_params,
)
def _extract_kernel(users_hbm, items_hbm, ult_hbm, ilt_hbm,
                    u_out, i_out, stage_v, pv_v, wl_v,
                    win_v, rows_v, gsem, ssem0, ssem1):
    wid = lax.axis_index("s") * NC + lax.axis_index("c")
    lo_c = wid * RPW
    hi_c = jnp.minimum(lo_c + RPW, NBT)
    nwin = (hi_c - lo_c + WBLK - 1) // WBLK
    lo_u = lo_c * GCHUNK

    iota = lax.iota(jnp.int32, L)
    dump = jnp.int32(B) + iota

    for idx_hbm, tbl, dst in ((users_hbm, ult_hbm, u_out),
                              (items_hbm, ilt_hbm, i_out)):
        def wstart(w, buf, tbl=tbl):
            c0 = jnp.minimum(lo_c + w * WBLK, NBT - WBLK)
            off = pl.multiple_of(c0 * GCHUNK, GCHUNK)
            pltpu.async_copy(tbl.at[:, pl.ds(off, WW)], win_v.at[buf], gsem)

        wstart(0, 0)

        def f1outer(ch, n, idx_hbm=idx_hbm, iota=iota):
            src = lax.rem(wid + ch, NFCH) * FCH
            pltpu.sync_copy(idx_hbm.at[pl.ds(src, FCH)], stage_v)

            def f1(k, n, src=src):
                u = stage_v[pl.ds(k * L, L)]
                c = jax.lax.shift_right_logical(u, 7)
                m = (c >= lo_c) & (c < hi_c)
                b = src + k * L + iota
                packed = jax.lax.shift_left(u - lo_u, 14) | b
                plsc.store_compressed(pv_v.at[pl.ds(n, L)], packed, mask=m)
                return n + plsc.all_reduce_population_count(m)[0]

            return lax.fori_loop(0, FCH // L, f1, n, unroll=4)

        n = lax.fori_loop(0, NFCH, f1outer, jnp.int32(0))
        for q in range(4):
            pv_v[pl.ds(n + q * L, L)] = jnp.full((L,), -1, jnp.int32)
        nv4 = (n + 4 * L - 1) // (4 * L)

        def win_body(w, carry, tbl=tbl, dst=dst, nv4=nv4, iota=iota, dump=dump):
            pend0, pend1 = carry
            buf = lax.rem(w, 2)

            @pl.when(w + 1 < nwin)
            def _():
                wstart(w + 1, 1 - buf)

            wr0 = w * WBLK * GCHUNK

            def f2(k, wn):
                for q in range(4):
                    pv = pv_v[pl.ds(k * 4 * L + q * L, L)]
                    ur = jax.lax.shift_right_logical(pv, 14)
                    m = (pv >= 0) & (ur >= wr0) & (ur < wr0 + WW)
                    plsc.store_compressed(wl_v.at[pl.ds(wn, L)], pv, mask=m)
                    wn = wn + plsc.all_reduce_population_count(m)[0]
                return wn

            wn = lax.fori_loop(0, nv4, f2, jnp.int32(0))
            wl_v[pl.ds(wn, L)] = jnp.full((L,), -1, jnp.int32)
            ngrp = (wn + L - 1) // L

            pltpu.make_async_copy(
                tbl.at[:, pl.ds(0, WW)], win_v.at[0], gsem).wait()

            c0 = jnp.minimum(lo_c + w * WBLK, NBT - WBLK)
            wloc0 = (c0 - lo_c) * GCHUNK

            def g_body(g, pends, buf=buf, wloc0=wloc0, dst=dst):
                p0, p1 = pends
                rbuf = lax.rem(g, 2)
                pv = wl_v[pl.ds(g * L, L)]
                bvec = pv & 0x3FFF
                ur = jax.lax.shift_right_logical(pv, 14)
                bsafe = jnp.where(pv < 0, dump, bvec)
                ulocal = jnp.where(pv < 0, 0, ur - wloc0)

                @pl.when((rbuf == 0) & (p0 > 0))
                def _():
                    pltpu.make_async_copy(
                        dst.at[pl.ds(0, L)], rows_v.at[0], ssem0).wait()

                @pl.when((rbuf == 1) & (p1 > 0))
                def _():
                    pltpu.make_async_copy(
                        dst.at[pl.ds(0, L)], rows_v.at[1], ssem1).wait()

                for d in range(DIM):
                    dvec = jnp.full((L,), d, jnp.int32)
                    vals = plsc.load_gather(win_v.at[buf], [dvec, ulocal])
                    plsc.store_scatter(rows_v.at[rbuf], [iota, dvec], vals)

                @pl.when(rbuf == 0)
                def _():
                    pltpu.async_copy(rows_v.at[0], dst.at[bsafe], ssem0)

                @pl.when(rbuf == 1)
                def _():
                    pltpu.async_copy(rows_v.at[1], dst.at[bsafe], ssem1)

                new_p0 = jnp.where(rbuf == 0, jnp.int32(1), p0)
                new_p1 = jnp.where(rbuf == 1, jnp.int32(1), p1)
                return (new_p0, new_p1)

            return lax.fori_loop(0, ngrp, g_body, (pend0, pend1))

        pend0, pend1 = lax.fori_loop(
            0, nwin, win_body, (jnp.int32(0), jnp.int32(0)))

        @pl.when(pend0 > 0)
        def _():
            pltpu.make_async_copy(
                dst.at[pl.ds(0, L)], rows_v.at[0], ssem0).wait()

        @pl.when(pend1 > 0)
        def _():
            pltpu.make_async_copy(
                dst.at[pl.ds(0, L)], rows_v.at[1], ssem1).wait()


@functools.partial(
    pl.kernel,
    out_type=jax.ShapeDtypeStruct((B,), jnp.float32),
    mesh=_mesh,
    scratch_types=[
        pltpu.VMEM((BPW,), jnp.int32),
        pltpu.VMEM((BPW,), jnp.int32),
        pltpu.VMEM((2, GCHUNK, GCHUNK), jnp.float32),
        pltpu.VMEM((2, GCHUNK, GCHUNK), jnp.float32),
        pltpu.VMEM((BPW,), jnp.float32),
        pltpu.VMEM((BPW,), jnp.float32),
        pltpu.VMEM((BPW,), jnp.float32),
        pltpu.SemaphoreType.DMA,
        pltpu.SemaphoreType.DMA,
    ],
    compiler_params=_sc_params,
)
def _dot_kernel(users_hbm, items_hbm, u_hbm, i_hbm, ub_hbm, ib_hbm,
                out_hbm, uidx_v, iidx_v, usub_v, isub_v,
                ub_v, ib_v, out_v, sem, bsem):
    wid = lax.axis_index("s") * NC + lax.axis_index("c")
    base = wid * BPW
    iota = lax.iota(jnp.int32, L)

    pltpu.sync_copy(users_hbm.at[pl.ds(base, BPW)], uidx_v)
    pltpu.sync_copy(items_hbm.at[pl.ds(base, BPW)], iidx_v)

    bias_copies = []
    for g in range(NG):
        sl = pl.ds(g * GCHUNK, GCHUNK)
        bias_copies.append(
            pltpu.async_copy(ub_hbm.at[uidx_v.at[sl]], ub_v.at[sl], bsem))
        bias_copies.append(
            pltpu.async_copy(ib_hbm.at[iidx_v.at[sl]], ib_v.at[sl], bsem))

    def sstart(s, buf):
        sl = pl.ds(base + s * GCHUNK, GCHUNK)
        pltpu.async_copy(u_hbm.at[sl], usub_v.at[buf], sem)
        pltpu.async_copy(i_hbm.at[sl], isub_v.at[buf], sem)

    sstart(0, 0)
    for c in bias_copies:
        c.wait()

    def sub_body(s, _):
        buf = lax.rem(s, 2)
        pltpu.make_async_copy(
            u_hbm.at[pl.ds(0, GCHUNK)], usub_v.at[0], sem).wait()
        pltpu.make_async_copy(
            i_hbm.at[pl.ds(0, GCHUNK)], isub_v.at[0], sem).wait()

        @pl.when(s + 1 < NG)
        def _():
            sstart(s + 1, 1 - buf)

        def chunk(cc, _, buf=buf, s=s):
            j0 = s * GCHUNK + cc * L
            rows = cc * L + iota
            acc = ub_v[pl.ds(j0, L)] + ib_v[pl.ds(j0, L)]
            for d in range(DIM):
                dvec = jnp.full((L,), d, jnp.int32)
                uv = plsc.load_gather(usub_v.at[buf], [rows, dvec])
                iv = plsc.load_gather(isub_v.at[buf], [rows, dvec])
                acc = acc + uv * iv
            out_v[pl.ds(j0, L)] = acc
            return ()

        lax.fori_loop(0, GCHUNK // L, chunk, ())
        return ()

    lax.fori_loop(0, NG, sub_body, ())

    pltpu.sync_copy(out_v, out_hbm.at[pl.ds(base, BPW)])


def kernel(users, items, user_latent, item_latent, user_bias, item_bias):
    users = users.astype(jnp.int32)
    items = items.astype(jnp.int32)
    ub = user_bias.reshape(-1)
    ib = item_bias.reshape(-1)
    u_rows, i_rows = _extract_kernel(users, items, user_latent.T, item_latent.T)
    return _dot_kernel(users, items, u_rows, i_rows, ub, ib)
